# Initial kernel scaffold; baseline (speedup 1.0000x reference)
import functools

import jax
import jax.numpy as jnp
from jax import lax
from jax.experimental import pallas as pl
from jax.experimental.pallas import tpu as pltpu
from jax.experimental.pallas import tpu_sc as plsc

N = 10000
E = 320000
D1 = 128
D2 = 64
NC = 2
NS = 16
NW = NC * NS
NPAD = 10240
NR = NPAD // NS
EW = 80
EC = E // EW
ET = EC // NW
BM = 1000

_mesh = plsc.VectorSubcoreMesh(core_axis_name="c", subcore_axis_name="s")


@functools.partial(
    pl.kernel,
    out_type=jax.ShapeDtypeStruct((NC, NPAD), jnp.float32),
    mesh=_mesh,
    scratch_types=[
        pltpu.VMEM((ET, EW), jnp.int32),
        pltpu.VMEM((EW,), jnp.float32),
        pltpu.VMEM((NR,), jnp.float32),
        pltpu.VMEM_SHARED((NPAD,), jnp.float32),
    ],
)
def _deg(dst_hbm, out_hbm, di_v, ones_v, zero_v, acc):
    cid = lax.axis_index("c")
    sid = lax.axis_index("s")
    wid = cid * NS + sid

    def _fill(k, _):
        zero_v[pl.ds(k * 16, 16)] = jnp.zeros((16,), jnp.float32)
        return 0

    lax.fori_loop(0, NR // 16, _fill, 0)

    def _fill1(k, _):
        ones_v[pl.ds(k * 16, 16)] = jnp.ones((16,), jnp.float32)
        return 0

    lax.fori_loop(0, EW // 16, _fill1, 0)
    pltpu.sync_copy(zero_v, acc.at[pl.ds(sid * NR, NR)])
    pltpu.sync_copy(dst_hbm.at[pl.ds(wid * ET, ET)], di_v)
    plsc.subcore_barrier()

    def _scat(j, _):
        pltpu.sync_copy(ones_v, acc.at[di_v.at[j]], add=True)
        return 0

    lax.fori_loop(0, ET, _scat, 0)
    plsc.subcore_barrier()
    pltpu.sync_copy(acc.at[pl.ds(sid * NR, NR)],
                    out_hbm.at[cid, pl.ds(sid * NR, NR)])


def _make_agg(D):
    @functools.partial(
        pl.kernel,
        out_type=jax.ShapeDtypeStruct((NC, NPAD, D), jnp.float32),
        mesh=_mesh,
        scratch_types=[
            pltpu.VMEM((ET, EW), jnp.int32),
            pltpu.VMEM((ET, EW), jnp.int32),
            pltpu.VMEM((EW, D), jnp.float32),
            pltpu.VMEM((EW, D), jnp.float32),
            pltpu.VMEM((EW, D), jnp.float32),
            pltpu.SemaphoreType.DMA,
            pltpu.SemaphoreType.DMA,
            pltpu.VMEM_SHARED((NPAD, D), jnp.float32),
        ],
    )
    def _agg(y_hbm, src_hbm, dst_hbm, out_hbm,
             si_v, di_v, r0, r1, zb, sem0, sem1, acc):
        cid = lax.axis_index("c")
        sid = lax.axis_index("s")
        wid = cid * NS + sid

        def _zrow(i, _):
            def _zcol(j, _):
                zb[i, pl.ds(j * 16, 16)] = jnp.zeros((16,), jnp.float32)
                return 0

            lax.fori_loop(0, D // 16, _zcol, 0)
            return 0

        lax.fori_loop(0, EW, _zrow, 0)

        def _zcp(r, _):
            pltpu.sync_copy(zb, acc.at[pl.ds(sid * NR + r * EW, EW)])
            return 0

        lax.fori_loop(0, NR // EW, _zcp, 0)
        pltpu.sync_copy(src_hbm.at[pl.ds(wid * ET, ET)], si_v)
        pltpu.sync_copy(dst_hbm.at[pl.ds(wid * ET, ET)], di_v)
        plsc.subcore_barrier()

        pltpu.async_copy(y_hbm.at[si_v.at[0]], r0, sem0)

        def _step(t, _):
            j = 2 * t
            pltpu.async_copy(y_hbm.at[si_v.at[j + 1]], r1, sem1)
            pltpu.make_async_copy(y_hbm.at[si_v.at[j]], r0, sem0).wait()
            pltpu.sync_copy(r0, acc.at[di_v.at[j]], add=True)
            pltpu.async_copy(y_hbm.at[si_v.at[j + 2]], r0, sem0)
            pltpu.make_async_copy(y_hbm.at[si_v.at[j + 1]], r1, sem1).wait()
            pltpu.sync_copy(r1, acc.at[di_v.at[j + 1]], add=True)
            return 0

        lax.fori_loop(0, (ET - 1) // 2, _step, 0)
        pltpu.make_async_copy(y_hbm.at[si_v.at[ET - 1]], r0, sem0).wait()
        pltpu.sync_copy(r0, acc.at[di_v.at[ET - 1]], add=True)
        plsc.subcore_barrier()
        pltpu.sync_copy(acc.at[pl.ds(sid * NR, NR)],
                        out_hbm.at[cid, pl.ds(sid * NR, NR)])

    return _agg


_agg128 = _make_agg(D1)
_agg64 = _make_agg(D2)


def _mm_body(x_ref, w_ref, o_ref):
    o_ref[...] = jnp.dot(x_ref[...], w_ref[...],
                         preferred_element_type=jnp.float32)


_mm1 = pl.pallas_call(
    _mm_body,
    grid=(N // BM,),
    in_specs=[pl.BlockSpec((BM, D1), lambda i: (i, 0)),
              pl.BlockSpec((D1, D1), lambda i: (0, 0))],
    out_specs=pl.BlockSpec((BM, D1), lambda i: (i, 0)),
    out_shape=jax.ShapeDtypeStruct((N, D1), jnp.float32),
)


def _y1_body(xw_ref, g0_ref, g1_ref, o_ref):
    dinv = lax.rsqrt(g0_ref[...] + g1_ref[...] + 1.0)
    o_ref[...] = dinv * xw_ref[...]


_y1 = pl.pallas_call(
    _y1_body,
    grid=(N // BM,),
    in_specs=[pl.BlockSpec((BM, D1), lambda i: (i, 0)),
              pl.BlockSpec((BM, 1), lambda i: (i, 0)),
              pl.BlockSpec((BM, 1), lambda i: (i, 0))],
    out_specs=pl.BlockSpec((BM, D1), lambda i: (i, 0)),
    out_shape=jax.ShapeDtypeStruct((N, D1), jnp.float32),
)


def _h_body(p0_ref, p1_ref, y1_ref, g0_ref, g1_ref, b1_ref, w2_ref, o_ref):
    dinv = lax.rsqrt(g0_ref[...] + g1_ref[...] + 1.0)
    h = jnp.maximum(
        dinv * (p0_ref[...] + p1_ref[...] + y1_ref[...]) + b1_ref[...], 0.0)
    o_ref[...] = dinv * jnp.dot(h, w2_ref[...],
                                preferred_element_type=jnp.float32)


_h = pl.pallas_call(
    _h_body,
    grid=(N // BM,),
    in_specs=[pl.BlockSpec((BM, D1), lambda i: (i, 0)),
              pl.BlockSpec((BM, D1), lambda i: (i, 0)),
              pl.BlockSpec((BM, D1), lambda i: (i, 0)),
              pl.BlockSpec((BM, 1), lambda i: (i, 0)),
              pl.BlockSpec((BM, 1), lambda i: (i, 0)),
              pl.BlockSpec((1, D1), lambda i: (0, 0)),
              pl.BlockSpec((D1, D2), lambda i: (0, 0))],
    out_specs=pl.BlockSpec((BM, D2), lambda i: (i, 0)),
    out_shape=jax.ShapeDtypeStruct((N, D2), jnp.float32),
)


def _z_body(q0_ref, q1_ref, y2_ref, g0_ref, g1_ref, b2_ref, o_ref):
    dinv = lax.rsqrt(g0_ref[...] + g1_ref[...] + 1.0)
    o_ref[...] = dinv * (q0_ref[...] + q1_ref[...] + y2_ref[...]) + b2_ref[...]


_z = pl.pallas_call(
    _z_body,
    grid=(N // BM,),
    in_specs=[pl.BlockSpec((BM, D2), lambda i: (i, 0)),
              pl.BlockSpec((BM, D2), lambda i: (i, 0)),
              pl.BlockSpec((BM, D2), lambda i: (i, 0)),
              pl.BlockSpec((BM, 1), lambda i: (i, 0)),
              pl.BlockSpec((BM, 1), lambda i: (i, 0)),
              pl.BlockSpec((1, D2), lambda i: (0, 0))],
    out_specs=pl.BlockSpec((BM, D2), lambda i: (i, 0)),
    out_shape=jax.ShapeDtypeStruct((N, D2), jnp.float32),
)


def kernel(x, edge_index, W1, b1, W2, b2):
    ei = edge_index.astype(jnp.int32)
    src2 = ei[0].reshape(EC, EW)
    dst2 = ei[1].reshape(EC, EW)

    degp = _deg(dst2)
    xw1 = _mm1(x, W1)
    g0 = degp[0][:, None]
    g1 = degp[1][:, None]
    y1 = _y1(xw1, g0, g1)
    P = _agg128(y1, src2, dst2)
    y2 = _h(P[0], P[1], y1, g0, g1, b1[None, :], W2)
    Q = _agg64(y2, src2, dst2)
    return _z(Q[0], Q[1], y2, g0, g1, b2[None, :])


# speedup vs baseline: 30.0630x; 30.0630x over previous
import functools

import jax
import jax.numpy as jnp
from jax import lax
from jax.experimental import pallas as pl
from jax.experimental.pallas import tpu as pltpu
from jax.experimental.pallas import tpu_sc as plsc

N = 10000
E = 320000
D1 = 128
D2 = 64
NC = 2
NS = 16
NW = NC * NS
NPAD = 10240
NR = NPAD // NS
EW = 125
EC = E // EW
ET = EC // NW
ZR = 32
IG = 16
BM = 1000

_mesh = plsc.VectorSubcoreMesh(core_axis_name="c", subcore_axis_name="s")


@functools.partial(
    pl.kernel,
    out_type=jax.ShapeDtypeStruct((NC * NPAD,), jnp.float32),
    mesh=_mesh,
    scratch_types=[
        pltpu.VMEM((ET, EW), jnp.int32),
        pltpu.VMEM((128,), jnp.float32),
        pltpu.VMEM((NR,), jnp.float32),
        pltpu.VMEM_SHARED((NPAD,), jnp.float32),
    ],
)
def _deg(dst_hbm, out_hbm, di_v, ones_v, zero_v, acc):
    cid = lax.axis_index("c")
    sid = lax.axis_index("s")
    wid = cid * NS + sid

    def _fill(k, _):
        zero_v[pl.ds(k * 16, 16)] = jnp.zeros((16,), jnp.float32)
        return 0

    lax.fori_loop(0, NR // 16, _fill, 0)

    def _fill1(k, _):
        ones_v[pl.ds(k * 16, 16)] = jnp.ones((16,), jnp.float32)
        return 0

    lax.fori_loop(0, 128 // 16, _fill1, 0)
    pltpu.sync_copy(zero_v, acc.at[pl.ds(sid * NR, NR)])
    pltpu.sync_copy(dst_hbm.at[pl.ds(wid * ET, ET)], di_v)
    plsc.subcore_barrier()

    def _scat(j, _):
        pltpu.sync_copy(ones_v.at[pl.ds(0, EW)], acc.at[di_v.at[j]], add=True)
        return 0

    lax.fori_loop(0, ET, _scat, 0)
    plsc.subcore_barrier()
    pltpu.sync_copy(acc.at[pl.ds(sid * NR, NR)],
                    out_hbm.at[pl.ds(cid * NPAD + sid * NR, NR)])


def _make_agg(D):
    @functools.partial(
        pl.kernel,
        out_type=jax.ShapeDtypeStruct((NC, NPAD, D), jnp.float32),
        mesh=_mesh,
        scratch_types=[
            pltpu.VMEM((IG, EW), jnp.int32),
            pltpu.VMEM((IG, EW), jnp.int32),
            pltpu.VMEM((IG, EW), jnp.int32),
            pltpu.VMEM((IG, EW), jnp.int32),
            pltpu.VMEM((EW, D), jnp.float32),
            pltpu.VMEM((EW, D), jnp.float32),
            pltpu.VMEM((ZR, D), jnp.float32),
            pltpu.SemaphoreType.DMA,
            pltpu.SemaphoreType.DMA,
            pltpu.SemaphoreType.DMA,
            pltpu.SemaphoreType.DMA,
            pltpu.VMEM_SHARED((NPAD, D), jnp.float32),
        ],
    )
    def _agg(y_hbm, src_hbm, dst_hbm, out_hbm,
             si_a, si_b, di_a, di_b, r0, r1, zb,
             sem0, sem1, sem_s, sem_d, acc):
        cid = lax.axis_index("c")
        sid = lax.axis_index("s")
        wid = cid * NS + sid

        def _zrow(i, _):
            def _zcol(j, _):
                zb[i, pl.ds(j * 16, 16)] = jnp.zeros((16,), jnp.float32)
                return 0

            lax.fori_loop(0, D // 16, _zcol, 0)
            return 0

        lax.fori_loop(0, ZR, _zrow, 0)

        def _zcp(r, _):
            pltpu.sync_copy(zb, acc.at[pl.ds(sid * NR + r * ZR, ZR)])
            return 0

        lax.fori_loop(0, NR // ZR, _zcp, 0)
        plsc.subcore_barrier()

        ebase = wid * ET
        pltpu.async_copy(src_hbm.at[pl.ds(ebase, IG)], si_a, sem_s)
        pltpu.async_copy(dst_hbm.at[pl.ds(ebase, IG)], di_a, sem_d)
        for g in range(ET // IG):
            si, di = (si_a, di_a) if g % 2 == 0 else (si_b, di_b)
            sn, dn = (si_b, di_b) if g % 2 == 0 else (si_a, di_a)
            pltpu.make_async_copy(src_hbm.at[pl.ds(ebase, IG)], si, sem_s).wait()
            pltpu.make_async_copy(dst_hbm.at[pl.ds(ebase, IG)], di, sem_d).wait()
            if g + 1 < ET // IG:
                off = ebase + (g + 1) * IG
                pltpu.async_copy(src_hbm.at[pl.ds(off, IG)], sn, sem_s)
                pltpu.async_copy(dst_hbm.at[pl.ds(off, IG)], dn, sem_d)
            pltpu.async_copy(y_hbm.at[si.at[0]], r0, sem0)

            def _step(t, _):
                j = 2 * t
                pltpu.async_copy(y_hbm.at[si.at[j + 1]], r1, sem1)
                pltpu.make_async_copy(y_hbm.at[si.at[j]], r0, sem0).wait()
                pltpu.sync_copy(r0, acc.at[di.at[j]], add=True)
                pltpu.async_copy(y_hbm.at[si.at[j + 2]], r0, sem0)
                pltpu.make_async_copy(y_hbm.at[si.at[j + 1]], r1, sem1).wait()
                pltpu.sync_copy(r1, acc.at[di.at[j + 1]], add=True)
                return 0

            lax.fori_loop(0, IG // 2 - 1, _step, 0)
            pltpu.async_copy(y_hbm.at[si.at[IG - 1]], r1, sem1)
            pltpu.make_async_copy(y_hbm.at[si.at[IG - 2]], r0, sem0).wait()
            pltpu.sync_copy(r0, acc.at[di.at[IG - 2]], add=True)
            pltpu.make_async_copy(y_hbm.at[si.at[IG - 1]], r1, sem1).wait()
            pltpu.sync_copy(r1, acc.at[di.at[IG - 1]], add=True)
        plsc.subcore_barrier()
        pltpu.sync_copy(acc.at[pl.ds(sid * NR, NR)],
                        out_hbm.at[cid, pl.ds(sid * NR, NR)])

    return _agg


_agg128 = _make_agg(D1)


def _mm_body(x_ref, w_ref, o_ref):
    o_ref[...] = jnp.dot(x_ref[...], w_ref[...],
                         preferred_element_type=jnp.float32)


_mm1 = pl.pallas_call(
    _mm_body,
    grid=(N // BM,),
    in_specs=[pl.BlockSpec((BM, D1), lambda i: (i, 0)),
              pl.BlockSpec((D1, D1), lambda i: (0, 0))],
    out_specs=pl.BlockSpec((BM, D1), lambda i: (i, 0)),
    out_shape=jax.ShapeDtypeStruct((N, D1), jnp.float32),
)


def _y1_body(xw_ref, g0_ref, g1_ref, o_ref):
    dinv = lax.rsqrt(g0_ref[...] + g1_ref[...] + 1.0)
    o_ref[...] = dinv * xw_ref[...]


_y1 = pl.pallas_call(
    _y1_body,
    grid=(N // BM,),
    in_specs=[pl.BlockSpec((BM, D1), lambda i: (i, 0)),
              pl.BlockSpec((BM, 1), lambda i: (i, 0)),
              pl.BlockSpec((BM, 1), lambda i: (i, 0))],
    out_specs=pl.BlockSpec((BM, D1), lambda i: (i, 0)),
    out_shape=jax.ShapeDtypeStruct((N, D1), jnp.float32),
)


def _h_body(p0_ref, p1_ref, y1_ref, g0_ref, g1_ref, b1_ref, o_ref):
    dinv = lax.rsqrt(g0_ref[...] + g1_ref[...] + 1.0)
    h = jnp.maximum(
        dinv * (p0_ref[...] + p1_ref[...] + y1_ref[...]) + b1_ref[...], 0.0)
    o_ref[...] = dinv * h


_h = pl.pallas_call(
    _h_body,
    grid=(N // BM,),
    in_specs=[pl.BlockSpec((BM, D1), lambda i: (i, 0)),
              pl.BlockSpec((BM, D1), lambda i: (i, 0)),
              pl.BlockSpec((BM, D1), lambda i: (i, 0)),
              pl.BlockSpec((BM, 1), lambda i: (i, 0)),
              pl.BlockSpec((BM, 1), lambda i: (i, 0)),
              pl.BlockSpec((1, D1), lambda i: (0, 0))],
    out_specs=pl.BlockSpec((BM, D1), lambda i: (i, 0)),
    out_shape=jax.ShapeDtypeStruct((N, D1), jnp.float32),
)


def _z_body(q0_ref, q1_ref, u_ref, g0_ref, g1_ref, b2_ref, w2_ref, o_ref):
    dinv = lax.rsqrt(g0_ref[...] + g1_ref[...] + 1.0)
    s = q0_ref[...] + q1_ref[...] + u_ref[...]
    o_ref[...] = dinv * jnp.dot(s, w2_ref[...],
                                preferred_element_type=jnp.float32) + b2_ref[...]


_z = pl.pallas_call(
    _z_body,
    grid=(N // BM,),
    in_specs=[pl.BlockSpec((BM, D1), lambda i: (i, 0)),
              pl.BlockSpec((BM, D1), lambda i: (i, 0)),
              pl.BlockSpec((BM, D1), lambda i: (i, 0)),
              pl.BlockSpec((BM, 1), lambda i: (i, 0)),
              pl.BlockSpec((BM, 1), lambda i: (i, 0)),
              pl.BlockSpec((1, D2), lambda i: (0, 0)),
              pl.BlockSpec((D1, D2), lambda i: (0, 0))],
    out_specs=pl.BlockSpec((BM, D2), lambda i: (i, 0)),
    out_shape=jax.ShapeDtypeStruct((N, D2), jnp.float32),
)


def kernel(x, edge_index, W1, b1, W2, b2):
    ei = edge_index.astype(jnp.int32)
    src2 = ei[0].reshape(EC, EW)
    dst2 = ei[1].reshape(EC, EW)

    degp = _deg(dst2)
    xw1 = _mm1(x, W1)
    g0 = degp[:NPAD, None]
    g1 = degp[NPAD:, None]
    y1 = _y1(xw1, g0, g1)
    P = _agg128(y1, src2, dst2)
    u = _h(P[0], P[1], y1, g0, g1, b1[None, :])
    Q = _agg128(u, src2, dst2)
    return _z(Q[0], Q[1], u, g0, g1, b2[None, :], W2)


# speedup vs baseline: 30.1601x; 1.0032x over previous
import functools

import jax
import jax.numpy as jnp
from jax import lax
from jax.experimental import pallas as pl
from jax.experimental.pallas import tpu as pltpu
from jax.experimental.pallas import tpu_sc as plsc

N = 10000
E = 320000
D1 = 128
D2 = 64
NC = 2
NS = 16
NW = NC * NS
NPAD = 10240
NR = NPAD // NS
EW = 125
EC = E // EW
ET = EC // NW
ZR = 32
IG = 16
BM = 1000

_mesh = plsc.VectorSubcoreMesh(core_axis_name="c", subcore_axis_name="s")


@functools.partial(
    pl.kernel,
    out_type=jax.ShapeDtypeStruct((NC * NPAD,), jnp.float32),
    mesh=_mesh,
    scratch_types=[
        pltpu.VMEM((ET, EW), jnp.int32),
        pltpu.VMEM((128,), jnp.float32),
        pltpu.VMEM((NR,), jnp.float32),
        pltpu.VMEM_SHARED((NPAD,), jnp.float32),
    ],
)
def _deg(dst_hbm, out_hbm, di_v, ones_v, zero_v, acc):
    cid = lax.axis_index("c")
    sid = lax.axis_index("s")
    wid = cid * NS + sid

    def _fill(k, _):
        zero_v[pl.ds(k * 16, 16)] = jnp.zeros((16,), jnp.float32)
        return 0

    lax.fori_loop(0, NR // 16, _fill, 0)

    def _fill1(k, _):
        ones_v[pl.ds(k * 16, 16)] = jnp.ones((16,), jnp.float32)
        return 0

    lax.fori_loop(0, 128 // 16, _fill1, 0)
    pltpu.sync_copy(zero_v, acc.at[pl.ds(sid * NR, NR)])
    pltpu.sync_copy(dst_hbm.at[pl.ds(wid * ET, ET)], di_v)
    plsc.subcore_barrier()

    def _scat(j, _):
        pltpu.sync_copy(ones_v.at[pl.ds(0, EW)], acc.at[di_v.at[j]], add=True)
        return 0

    lax.fori_loop(0, ET, _scat, 0)
    plsc.subcore_barrier()
    pltpu.sync_copy(acc.at[pl.ds(sid * NR, NR)],
                    out_hbm.at[pl.ds(cid * NPAD + sid * NR, NR)])


def _make_agg(D):
    @functools.partial(
        pl.kernel,
        out_type=jax.ShapeDtypeStruct((NC, NPAD, D), jnp.float32),
        mesh=_mesh,
        scratch_types=[
            pltpu.VMEM((IG, EW), jnp.int32),
            pltpu.VMEM((IG, EW), jnp.int32),
            pltpu.VMEM((IG, EW), jnp.int32),
            pltpu.VMEM((IG, EW), jnp.int32),
            pltpu.VMEM((EW, D), jnp.float32),
            pltpu.VMEM((EW, D), jnp.float32),
            pltpu.VMEM((ZR, D), jnp.float32),
            pltpu.SemaphoreType.DMA,
            pltpu.SemaphoreType.DMA,
            pltpu.SemaphoreType.DMA,
            pltpu.SemaphoreType.DMA,
            pltpu.VMEM_SHARED((NPAD, D), jnp.float32),
        ],
    )
    def _agg(y_hbm, src_hbm, dst_hbm, out_hbm,
             si_a, si_b, di_a, di_b, r0, r1, zb,
             sem0, sem1, sem_s, sem_d, acc):
        cid = lax.axis_index("c")
        sid = lax.axis_index("s")
        wid = cid * NS + sid

        def _zrow(i, _):
            def _zcol(j, _):
                zb[i, pl.ds(j * 16, 16)] = jnp.zeros((16,), jnp.float32)
                return 0

            lax.fori_loop(0, D // 16, _zcol, 0)
            return 0

        lax.fori_loop(0, ZR, _zrow, 0)

        def _zcp(r, _):
            pltpu.sync_copy(zb, acc.at[pl.ds(sid * NR + r * ZR, ZR)])
            return 0

        lax.fori_loop(0, NR // ZR, _zcp, 0)
        plsc.subcore_barrier()

        ebase = wid * ET
        pltpu.async_copy(src_hbm.at[pl.ds(ebase, IG)], si_a, sem_s)
        pltpu.async_copy(dst_hbm.at[pl.ds(ebase, IG)], di_a, sem_d)
        for g in range(ET // IG):
            si, di = (si_a, di_a) if g % 2 == 0 else (si_b, di_b)
            sn, dn = (si_b, di_b) if g % 2 == 0 else (si_a, di_a)
            pltpu.make_async_copy(src_hbm.at[pl.ds(ebase, IG)], si, sem_s).wait()
            pltpu.make_async_copy(dst_hbm.at[pl.ds(ebase, IG)], di, sem_d).wait()
            if g + 1 < ET // IG:
                off = ebase + (g + 1) * IG
                pltpu.async_copy(src_hbm.at[pl.ds(off, IG)], sn, sem_s)
                pltpu.async_copy(dst_hbm.at[pl.ds(off, IG)], dn, sem_d)
            pltpu.async_copy(y_hbm.at[si.at[0]], r0, sem0)

            def _step(t, _):
                j = 2 * t
                pltpu.async_copy(y_hbm.at[si.at[j + 1]], r1, sem1)
                pltpu.make_async_copy(y_hbm.at[si.at[j]], r0, sem0).wait()
                pltpu.sync_copy(r0, acc.at[di.at[j]], add=True)
                pltpu.async_copy(y_hbm.at[si.at[j + 2]], r0, sem0)
                pltpu.make_async_copy(y_hbm.at[si.at[j + 1]], r1, sem1).wait()
                pltpu.sync_copy(r1, acc.at[di.at[j + 1]], add=True)
                return 0

            lax.fori_loop(0, IG // 2 - 1, _step, 0)
            pltpu.async_copy(y_hbm.at[si.at[IG - 1]], r1, sem1)
            pltpu.make_async_copy(y_hbm.at[si.at[IG - 2]], r0, sem0).wait()
            pltpu.sync_copy(r0, acc.at[di.at[IG - 2]], add=True)
            pltpu.make_async_copy(y_hbm.at[si.at[IG - 1]], r1, sem1).wait()
            pltpu.sync_copy(r1, acc.at[di.at[IG - 1]], add=True)
        plsc.subcore_barrier()
        pltpu.sync_copy(acc.at[pl.ds(sid * NR, NR)],
                        out_hbm.at[cid, pl.ds(sid * NR, NR)])

    return _agg


_agg128 = _make_agg(D1)


def _y1_body(x_ref, w_ref, g0_ref, g1_ref, o_ref):
    dinv = lax.rsqrt(g0_ref[...] + g1_ref[...] + 1.0)
    o_ref[...] = dinv * jnp.dot(x_ref[...], w_ref[...],
                                preferred_element_type=jnp.float32)


_y1 = pl.pallas_call(
    _y1_body,
    grid=(N // BM,),
    in_specs=[pl.BlockSpec((BM, D1), lambda i: (i, 0)),
              pl.BlockSpec((D1, D1), lambda i: (0, 0)),
              pl.BlockSpec((BM, 1), lambda i: (i, 0)),
              pl.BlockSpec((BM, 1), lambda i: (i, 0))],
    out_specs=pl.BlockSpec((BM, D1), lambda i: (i, 0)),
    out_shape=jax.ShapeDtypeStruct((N, D1), jnp.float32),
)


def _h_body(p0_ref, p1_ref, y1_ref, g0_ref, g1_ref, b1_ref, o_ref):
    dinv = lax.rsqrt(g0_ref[...] + g1_ref[...] + 1.0)
    h = jnp.maximum(
        dinv * (p0_ref[...] + p1_ref[...] + y1_ref[...]) + b1_ref[...], 0.0)
    o_ref[...] = dinv * h


_h = pl.pallas_call(
    _h_body,
    grid=(N // BM,),
    in_specs=[pl.BlockSpec((BM, D1), lambda i: (i, 0)),
              pl.BlockSpec((BM, D1), lambda i: (i, 0)),
              pl.BlockSpec((BM, D1), lambda i: (i, 0)),
              pl.BlockSpec((BM, 1), lambda i: (i, 0)),
              pl.BlockSpec((BM, 1), lambda i: (i, 0)),
              pl.BlockSpec((1, D1), lambda i: (0, 0))],
    out_specs=pl.BlockSpec((BM, D1), lambda i: (i, 0)),
    out_shape=jax.ShapeDtypeStruct((N, D1), jnp.float32),
)


def _z_body(q0_ref, q1_ref, u_ref, g0_ref, g1_ref, b2_ref, w2_ref, o_ref):
    dinv = lax.rsqrt(g0_ref[...] + g1_ref[...] + 1.0)
    s = q0_ref[...] + q1_ref[...] + u_ref[...]
    o_ref[...] = dinv * jnp.dot(s, w2_ref[...],
                                preferred_element_type=jnp.float32) + b2_ref[...]


_z = pl.pallas_call(
    _z_body,
    grid=(N // BM,),
    in_specs=[pl.BlockSpec((BM, D1), lambda i: (i, 0)),
              pl.BlockSpec((BM, D1), lambda i: (i, 0)),
              pl.BlockSpec((BM, D1), lambda i: (i, 0)),
              pl.BlockSpec((BM, 1), lambda i: (i, 0)),
              pl.BlockSpec((BM, 1), lambda i: (i, 0)),
              pl.BlockSpec((1, D2), lambda i: (0, 0)),
              pl.BlockSpec((D1, D2), lambda i: (0, 0))],
    out_specs=pl.BlockSpec((BM, D2), lambda i: (i, 0)),
    out_shape=jax.ShapeDtypeStruct((N, D2), jnp.float32),
)


def kernel(x, edge_index, W1, b1, W2, b2):
    ei = edge_index.astype(jnp.int32)
    src2 = ei[0].reshape(EC, EW)
    dst2 = ei[1].reshape(EC, EW)

    degp = _deg(dst2)
    g0 = degp[:NPAD, None]
    g1 = degp[NPAD:, None]
    y1 = _y1(x, W1, g0, g1)
    P = _agg128(y1, src2, dst2)
    u = _h(P[0], P[1], y1, g0, g1, b1[None, :])
    Q = _agg128(u, src2, dst2)
    return _z(Q[0], Q[1], u, g0, g1, b2[None, :], W2)


# speedup vs baseline: 31.5071x; 1.0447x over previous
import functools

import jax
import jax.numpy as jnp
from jax import lax
from jax.experimental import pallas as pl
from jax.experimental.pallas import tpu as pltpu
from jax.experimental.pallas import tpu_sc as plsc

N = 10000
E = 320000
D1 = 128
D2 = 64
NC = 2
NS = 16
NW = NC * NS
NPAD = 10240
NR = NPAD // NS
EW = 125
EC = E // EW
ET = EC // NW
ZR = 32
IG = 16
BM = 1000

_mesh = plsc.VectorSubcoreMesh(core_axis_name="c", subcore_axis_name="s")


@functools.partial(
    pl.kernel,
    out_type=jax.ShapeDtypeStruct((NC * NPAD,), jnp.float32),
    mesh=_mesh,
    scratch_types=[
        pltpu.VMEM((ET, EW), jnp.int32),
        pltpu.VMEM((128,), jnp.float32),
        pltpu.VMEM((NR,), jnp.float32),
        pltpu.VMEM_SHARED((NPAD,), jnp.float32),
    ],
)
def _deg(dst_hbm, out_hbm, di_v, ones_v, zero_v, acc):
    cid = lax.axis_index("c")
    sid = lax.axis_index("s")
    wid = cid * NS + sid

    def _fill(k, _):
        zero_v[pl.ds(k * 16, 16)] = jnp.zeros((16,), jnp.float32)
        return 0

    lax.fori_loop(0, NR // 16, _fill, 0)

    def _fill1(k, _):
        ones_v[pl.ds(k * 16, 16)] = jnp.ones((16,), jnp.float32)
        return 0

    lax.fori_loop(0, 128 // 16, _fill1, 0)
    pltpu.sync_copy(zero_v, acc.at[pl.ds(sid * NR, NR)])
    pltpu.sync_copy(dst_hbm.at[pl.ds(wid * ET, ET)], di_v)
    plsc.subcore_barrier()

    def _scat(j, _):
        pltpu.sync_copy(ones_v.at[pl.ds(0, EW)], acc.at[di_v.at[j]], add=True)
        return 0

    lax.fori_loop(0, ET, _scat, 0)
    plsc.subcore_barrier()
    pltpu.sync_copy(acc.at[pl.ds(sid * NR, NR)],
                    out_hbm.at[pl.ds(cid * NPAD + sid * NR, NR)])


def _make_agg(D):
    @functools.partial(
        pl.kernel,
        out_type=jax.ShapeDtypeStruct((NC, NPAD, D), jnp.float32),
        mesh=_mesh,
        scratch_types=[
            pltpu.VMEM((IG, EW), jnp.int32),
            pltpu.VMEM((IG, EW), jnp.int32),
            pltpu.VMEM((IG, EW), jnp.int32),
            pltpu.VMEM((IG, EW), jnp.int32),
            pltpu.VMEM((EW, D), jnp.float32),
            pltpu.VMEM((EW, D), jnp.float32),
            pltpu.VMEM((ZR, D), jnp.float32),
            pltpu.SemaphoreType.DMA,
            pltpu.SemaphoreType.DMA,
            pltpu.SemaphoreType.DMA,
            pltpu.SemaphoreType.DMA,
            pltpu.VMEM_SHARED((NPAD, D), jnp.float32),
        ],
    )
    def _agg(y_hbm, src_hbm, dst_hbm, out_hbm,
             si_a, si_b, di_a, di_b, r0, r1, zb,
             sem0, sem1, sem_s, sem_d, acc):
        cid = lax.axis_index("c")
        sid = lax.axis_index("s")
        wid = cid * NS + sid

        def _zrow(i, _):
            def _zcol(j, _):
                zb[i, pl.ds(j * 16, 16)] = jnp.zeros((16,), jnp.float32)
                return 0

            lax.fori_loop(0, D // 16, _zcol, 0)
            return 0

        lax.fori_loop(0, ZR, _zrow, 0)

        def _zcp(r, _):
            pltpu.sync_copy(zb, acc.at[pl.ds(sid * NR + r * ZR, ZR)])
            return 0

        lax.fori_loop(0, NR // ZR, _zcp, 0)
        plsc.subcore_barrier()

        ebase = wid * ET
        pltpu.async_copy(src_hbm.at[pl.ds(ebase, IG)], si_a, sem_s)
        pltpu.async_copy(dst_hbm.at[pl.ds(ebase, IG)], di_a, sem_d)
        for g in range(ET // IG):
            si, di = (si_a, di_a) if g % 2 == 0 else (si_b, di_b)
            sn, dn = (si_b, di_b) if g % 2 == 0 else (si_a, di_a)
            pltpu.make_async_copy(src_hbm.at[pl.ds(ebase, IG)], si, sem_s).wait()
            pltpu.make_async_copy(dst_hbm.at[pl.ds(ebase, IG)], di, sem_d).wait()
            if g + 1 < ET // IG:
                off = ebase + (g + 1) * IG
                pltpu.async_copy(src_hbm.at[pl.ds(off, IG)], sn, sem_s)
                pltpu.async_copy(dst_hbm.at[pl.ds(off, IG)], dn, sem_d)
            pltpu.async_copy(y_hbm.at[si.at[0]], r0, sem0)

            def _step(t, _):
                j = 2 * t
                pltpu.async_copy(y_hbm.at[si.at[j + 1]], r1, sem1)
                pltpu.make_async_copy(y_hbm.at[si.at[j]], r0, sem0).wait()
                pltpu.sync_copy(r0, acc.at[di.at[j]], add=True)
                pltpu.async_copy(y_hbm.at[si.at[j + 2]], r0, sem0)
                pltpu.make_async_copy(y_hbm.at[si.at[j + 1]], r1, sem1).wait()
                pltpu.sync_copy(r1, acc.at[di.at[j + 1]], add=True)
                return 0

            lax.fori_loop(0, IG // 2 - 1, _step, 0)
            pltpu.async_copy(y_hbm.at[si.at[IG - 1]], r1, sem1)
            pltpu.make_async_copy(y_hbm.at[si.at[IG - 2]], r0, sem0).wait()
            pltpu.sync_copy(r0, acc.at[di.at[IG - 2]], add=True)
            pltpu.make_async_copy(y_hbm.at[si.at[IG - 1]], r1, sem1).wait()
            pltpu.sync_copy(r1, acc.at[di.at[IG - 1]], add=True)
        plsc.subcore_barrier()
        pltpu.sync_copy(acc.at[pl.ds(sid * NR, NR)],
                        out_hbm.at[cid, pl.ds(sid * NR, NR)])

    return _agg


_agg128 = _make_agg(D1)


def _y1_body(x_ref, w_ref, g0_ref, g1_ref, o_ref):
    dinv = lax.rsqrt(g0_ref[...] + g1_ref[...] + 1.0)
    o_ref[...] = dinv * jnp.dot(x_ref[...], w_ref[...],
                                preferred_element_type=jnp.float32)


_y1 = pl.pallas_call(
    _y1_body,
    grid=(N // BM,),
    in_specs=[pl.BlockSpec((BM, D1), lambda i: (i, 0)),
              pl.BlockSpec((D1, D1), lambda i: (0, 0)),
              pl.BlockSpec((BM, 1), lambda i: (i, 0)),
              pl.BlockSpec((BM, 1), lambda i: (i, 0))],
    out_specs=pl.BlockSpec((BM, D1), lambda i: (i, 0)),
    out_shape=jax.ShapeDtypeStruct((N, D1), jnp.float32),
)


def _h_body(p_ref, y1_ref, g0_ref, g1_ref, b1_ref, o_ref):
    dinv = lax.rsqrt(g0_ref[...] + g1_ref[...] + 1.0)
    h = jnp.maximum(
        dinv * (p_ref[0] + p_ref[1] + y1_ref[...]) + b1_ref[...], 0.0)
    o_ref[...] = dinv * h


_h = pl.pallas_call(
    _h_body,
    grid=(N // BM,),
    in_specs=[pl.BlockSpec((NC, BM, D1), lambda i: (0, i, 0)),
              pl.BlockSpec((BM, D1), lambda i: (i, 0)),
              pl.BlockSpec((BM, 1), lambda i: (i, 0)),
              pl.BlockSpec((BM, 1), lambda i: (i, 0)),
              pl.BlockSpec((1, D1), lambda i: (0, 0))],
    out_specs=pl.BlockSpec((BM, D1), lambda i: (i, 0)),
    out_shape=jax.ShapeDtypeStruct((N, D1), jnp.float32),
)


def _z_body(q_ref, u_ref, g0_ref, g1_ref, b2_ref, w2_ref, o_ref):
    dinv = lax.rsqrt(g0_ref[...] + g1_ref[...] + 1.0)
    s = q_ref[0] + q_ref[1] + u_ref[...]
    o_ref[...] = dinv * jnp.dot(s, w2_ref[...],
                                preferred_element_type=jnp.float32) + b2_ref[...]


_z = pl.pallas_call(
    _z_body,
    grid=(N // BM,),
    in_specs=[pl.BlockSpec((NC, BM, D1), lambda i: (0, i, 0)),
              pl.BlockSpec((BM, D1), lambda i: (i, 0)),
              pl.BlockSpec((BM, 1), lambda i: (i, 0)),
              pl.BlockSpec((BM, 1), lambda i: (i, 0)),
              pl.BlockSpec((1, D2), lambda i: (0, 0)),
              pl.BlockSpec((D1, D2), lambda i: (0, 0))],
    out_specs=pl.BlockSpec((BM, D2), lambda i: (i, 0)),
    out_shape=jax.ShapeDtypeStruct((N, D2), jnp.float32),
)


def kernel(x, edge_index, W1, b1, W2, b2):
    ei = edge_index.astype(jnp.int32)
    src2 = ei[0].reshape(EC, EW)
    dst2 = ei[1].reshape(EC, EW)

    degp = _deg(dst2)
    g0 = degp[:NPAD, None]
    g1 = degp[NPAD:, None]
    y1 = _y1(x, W1, g0, g1)
    P = _agg128(y1, src2, dst2)
    u = _h(P, y1, g0, g1, b1[None, :])
    Q = _agg128(u, src2, dst2)
    return _z(Q, u, g0, g1, b2[None, :], W2)


# speedup vs baseline: 32.8849x; 1.0437x over previous
import functools

import jax
import jax.numpy as jnp
from jax import lax
from jax.experimental import pallas as pl
from jax.experimental.pallas import tpu as pltpu
from jax.experimental.pallas import tpu_sc as plsc

N = 10000
E = 320000
D1 = 128
D2 = 64
NC = 2
NS = 16
NW = NC * NS
NPAD = 10240
NR = NPAD // NS
EW = 125
EC = E // EW
ET = EC // NW
ZR = 32
IG = 16
BM = 1000

_mesh = plsc.VectorSubcoreMesh(core_axis_name="c", subcore_axis_name="s")


@functools.partial(
    pl.kernel,
    out_type=jax.ShapeDtypeStruct((NC * NPAD,), jnp.float32),
    mesh=_mesh,
    scratch_types=[
        pltpu.VMEM((ET, EW), jnp.int32),
        pltpu.VMEM((128,), jnp.float32),
        pltpu.VMEM((NR,), jnp.float32),
        pltpu.VMEM_SHARED((NPAD,), jnp.float32),
    ],
)
def _deg(dst_hbm, out_hbm, di_v, ones_v, zero_v, acc):
    cid = lax.axis_index("c")
    sid = lax.axis_index("s")
    wid = cid * NS + sid

    def _fill(k, _):
        zero_v[pl.ds(k * 16, 16)] = jnp.zeros((16,), jnp.float32)
        return 0

    lax.fori_loop(0, NR // 16, _fill, 0)

    def _fill1(k, _):
        ones_v[pl.ds(k * 16, 16)] = jnp.ones((16,), jnp.float32)
        return 0

    lax.fori_loop(0, 128 // 16, _fill1, 0)
    pltpu.sync_copy(zero_v, acc.at[pl.ds(sid * NR, NR)])
    pltpu.sync_copy(dst_hbm.at[pl.ds(wid * ET, ET)], di_v)
    plsc.subcore_barrier()

    def _scat(j, _):
        pltpu.sync_copy(ones_v.at[pl.ds(0, EW)], acc.at[di_v.at[j]], add=True)
        return 0

    lax.fori_loop(0, ET, _scat, 0)
    plsc.subcore_barrier()
    pltpu.sync_copy(acc.at[pl.ds(sid * NR, NR)],
                    out_hbm.at[pl.ds(cid * NPAD + sid * NR, NR)])


def _make_agg(D):
    @functools.partial(
        pl.kernel,
        out_type=jax.ShapeDtypeStruct((NC, NPAD, D), jnp.float32),
        mesh=_mesh,
        scratch_types=[
            pltpu.VMEM((IG, EW), jnp.int32),
            pltpu.VMEM((IG, EW), jnp.int32),
            pltpu.VMEM((IG, EW), jnp.int32),
            pltpu.VMEM((IG, EW), jnp.int32),
            pltpu.VMEM((EW, D), jnp.float32),
            pltpu.VMEM((EW, D), jnp.float32),
            pltpu.VMEM((ZR, D), jnp.float32),
            pltpu.SemaphoreType.DMA,
            pltpu.SemaphoreType.DMA,
            pltpu.SemaphoreType.DMA,
            pltpu.SemaphoreType.DMA,
            pltpu.VMEM_SHARED((NPAD, D), jnp.float32),
        ],
    )
    def _agg(y_hbm, src_hbm, dst_hbm, out_hbm,
             si_a, si_b, di_a, di_b, r0, r1, zb,
             sem0, sem1, sem_s, sem_d, acc):
        cid = lax.axis_index("c")
        sid = lax.axis_index("s")
        wid = cid * NS + sid

        def _zrow(i, _):
            def _zcol(j, _):
                zb[i, pl.ds(j * 16, 16)] = jnp.zeros((16,), jnp.float32)
                return 0

            lax.fori_loop(0, D // 16, _zcol, 0)
            return 0

        lax.fori_loop(0, ZR, _zrow, 0)

        ebase = wid * ET
        NG = ET // IG
        pltpu.async_copy(src_hbm.at[pl.ds(ebase, IG)], si_a, sem_s)
        pltpu.async_copy(dst_hbm.at[pl.ds(ebase, IG)], di_a, sem_d)
        pltpu.make_async_copy(src_hbm.at[pl.ds(ebase, IG)], si_a, sem_s).wait()
        pltpu.make_async_copy(dst_hbm.at[pl.ds(ebase, IG)], di_a, sem_d).wait()
        pltpu.async_copy(src_hbm.at[pl.ds(ebase + IG, IG)], si_b, sem_s)
        pltpu.async_copy(dst_hbm.at[pl.ds(ebase + IG, IG)], di_b, sem_d)
        pltpu.async_copy(y_hbm.at[si_a.at[0]], r0, sem0)

        def _zcp(r, _):
            pltpu.sync_copy(zb, acc.at[pl.ds(sid * NR + r * ZR, ZR)])
            return 0

        lax.fori_loop(0, NR // ZR, _zcp, 0)
        plsc.subcore_barrier()

        for g in range(NG):
            si, di = (si_a, di_a) if g % 2 == 0 else (si_b, di_b)
            sn, dn = (si_b, di_b) if g % 2 == 0 else (si_a, di_a)

            def _step(t, _, si=si, di=di):
                j = 2 * t
                pltpu.async_copy(y_hbm.at[si.at[j + 1]], r1, sem1)
                pltpu.make_async_copy(y_hbm.at[si.at[j]], r0, sem0).wait()
                pltpu.sync_copy(r0, acc.at[di.at[j]], add=True)
                pltpu.async_copy(y_hbm.at[si.at[j + 2]], r0, sem0)
                pltpu.make_async_copy(y_hbm.at[si.at[j + 1]], r1, sem1).wait()
                pltpu.sync_copy(r1, acc.at[di.at[j + 1]], add=True)
                return 0

            lax.fori_loop(0, IG // 2 - 1, _step, 0)
            pltpu.async_copy(y_hbm.at[si.at[IG - 1]], r1, sem1)
            pltpu.make_async_copy(y_hbm.at[si.at[IG - 2]], r0, sem0).wait()
            pltpu.sync_copy(r0, acc.at[di.at[IG - 2]], add=True)
            if g + 1 < NG:
                pltpu.make_async_copy(
                    src_hbm.at[pl.ds(ebase, IG)], sn, sem_s).wait()
                pltpu.make_async_copy(
                    dst_hbm.at[pl.ds(ebase, IG)], dn, sem_d).wait()
                pltpu.async_copy(y_hbm.at[sn.at[0]], r0, sem0)
            pltpu.make_async_copy(y_hbm.at[si.at[IG - 1]], r1, sem1).wait()
            pltpu.sync_copy(r1, acc.at[di.at[IG - 1]], add=True)
            if g + 2 < NG:
                off = ebase + (g + 2) * IG
                pltpu.async_copy(src_hbm.at[pl.ds(off, IG)], si, sem_s)
                pltpu.async_copy(dst_hbm.at[pl.ds(off, IG)], di, sem_d)
        plsc.subcore_barrier()
        pltpu.sync_copy(acc.at[pl.ds(sid * NR, NR)],
                        out_hbm.at[cid, pl.ds(sid * NR, NR)])

    return _agg


_agg128 = _make_agg(D1)


def _y1_body(x_ref, w_ref, g0_ref, g1_ref, o_ref):
    dinv = lax.rsqrt(g0_ref[...] + g1_ref[...] + 1.0)
    o_ref[...] = dinv * jnp.dot(x_ref[...], w_ref[...],
                                preferred_element_type=jnp.float32)


_y1 = pl.pallas_call(
    _y1_body,
    grid=(N // BM,),
    in_specs=[pl.BlockSpec((BM, D1), lambda i: (i, 0)),
              pl.BlockSpec((D1, D1), lambda i: (0, 0)),
              pl.BlockSpec((BM, 1), lambda i: (i, 0)),
              pl.BlockSpec((BM, 1), lambda i: (i, 0))],
    out_specs=pl.BlockSpec((BM, D1), lambda i: (i, 0)),
    out_shape=jax.ShapeDtypeStruct((N, D1), jnp.float32),
)


def _h_body(p_ref, y1_ref, g0_ref, g1_ref, b1_ref, o_ref):
    dinv = lax.rsqrt(g0_ref[...] + g1_ref[...] + 1.0)
    h = jnp.maximum(
        dinv * (p_ref[0] + p_ref[1] + y1_ref[...]) + b1_ref[...], 0.0)
    o_ref[...] = dinv * h


_h = pl.pallas_call(
    _h_body,
    grid=(N // BM,),
    in_specs=[pl.BlockSpec((NC, BM, D1), lambda i: (0, i, 0)),
              pl.BlockSpec((BM, D1), lambda i: (i, 0)),
              pl.BlockSpec((BM, 1), lambda i: (i, 0)),
              pl.BlockSpec((BM, 1), lambda i: (i, 0)),
              pl.BlockSpec((1, D1), lambda i: (0, 0))],
    out_specs=pl.BlockSpec((BM, D1), lambda i: (i, 0)),
    out_shape=jax.ShapeDtypeStruct((N, D1), jnp.float32),
)


def _z_body(q_ref, u_ref, g0_ref, g1_ref, b2_ref, w2_ref, o_ref):
    dinv = lax.rsqrt(g0_ref[...] + g1_ref[...] + 1.0)
    s = q_ref[0] + q_ref[1] + u_ref[...]
    o_ref[...] = dinv * jnp.dot(s, w2_ref[...],
                                preferred_element_type=jnp.float32) + b2_ref[...]


_z = pl.pallas_call(
    _z_body,
    grid=(N // BM,),
    in_specs=[pl.BlockSpec((NC, BM, D1), lambda i: (0, i, 0)),
              pl.BlockSpec((BM, D1), lambda i: (i, 0)),
              pl.BlockSpec((BM, 1), lambda i: (i, 0)),
              pl.BlockSpec((BM, 1), lambda i: (i, 0)),
              pl.BlockSpec((1, D2), lambda i: (0, 0)),
              pl.BlockSpec((D1, D2), lambda i: (0, 0))],
    out_specs=pl.BlockSpec((BM, D2), lambda i: (i, 0)),
    out_shape=jax.ShapeDtypeStruct((N, D2), jnp.float32),
)


def kernel(x, edge_index, W1, b1, W2, b2):
    ei = edge_index.astype(jnp.int32)
    src2 = ei[0].reshape(EC, EW)
    dst2 = ei[1].reshape(EC, EW)

    degp = _deg(dst2)
    g0 = degp[:NPAD, None]
    g1 = degp[NPAD:, None]
    y1 = _y1(x, W1, g0, g1)
    P = _agg128(y1, src2, dst2)
    u = _h(P, y1, g0, g1, b1[None, :])
    Q = _agg128(u, src2, dst2)
    return _z(Q, u, g0, g1, b2[None, :], W2)
